# baseline (speedup 1.0000x reference)
import jax
import jax.numpy as jnp
from jax import lax
from jax.experimental import pallas as pl
from jax.experimental.pallas import tpu as pltpu
from jax.experimental.pallas import tpu_sc as plsc

V = 1000001
D = 64
B = 16384
NUM_CORES = 2
NUM_SUBCORES = 16
NW = NUM_CORES * NUM_SUBCORES
SBW = 256
NSB_FULL = 7812 * 128 // SBW
TAIL_START = NSB_FULL * SBW
TAIL_W = V - TAIL_START
QS, RS = divmod(NSB_FULL, NW)
ROWCHUNK = 128
PAD_ROWS = B + NW * ROWCHUNK
NCHUNKS_IDX = B // 16
CNT_PAD = 128
CAP = 32
NBUF = 4


def _sweep_body(tt_hbm, idx_hbm, tail_hbm, out_hbm,
                idx_v, mlist, counts, glist2, buf0, buf1, buf2, buf3,
                tailbuf, rowbuf, blist, b2d,
                sem0, sem1, sem2, sem3, semt, sem_s):
    wid = lax.axis_index("s") * NUM_CORES + lax.axis_index("c")
    lo = wid * QS + jnp.minimum(wid, RS)
    n_sweep = QS + (wid < RS).astype(jnp.int32)
    has_tail = wid == NW - 1
    n_sb = n_sweep + has_tail.astype(jnp.int32)
    hi = lo + n_sb
    lanes = jnp.arange(16, dtype=jnp.int32)
    zeros16 = jnp.zeros((16,), jnp.int32)

    def dyn_read(ref, i):
        return plsc.load_gather(ref, [jnp.broadcast_to(i, (16,))])[0]

    def dyn_write(ref, i, val):
        plsc.store_scatter(ref, [jnp.broadcast_to(i, (16,))],
                           jnp.broadcast_to(val, (16,)), mask=lanes == 0)

    def fire(t, buf, sem):
        src = tt_hbm.at[:, pl.ds(pl.multiple_of((lo + t) * SBW, SBW), SBW)]
        pltpu.async_copy(src, buf, sem)

    fire(0, buf0, sem0)
    fire(1, buf1, sem1)
    fire(2, buf2, sem2)
    fire(3, buf3, sem3)

    @pl.when(has_tail)
    def _():
        pltpu.async_copy(tail_hbm, tailbuf, semt)

    pltpu.sync_copy(idx_hbm, idx_v.at[pl.ds(0, B)])

    def scan_step(k, ptr):
        v = idx_v[pl.ds(pl.multiple_of(k * 16, 16), 16)]
        sb = v >> 8
        m = (sb >= lo) & (sb < hi)
        col = v & (SBW - 1)
        bpos = k * 16 + lanes
        packed = ((sb - lo) << 22) | (col << 14) | bpos
        csum = plsc.cumsum(m.astype(jnp.int32))
        plsc.store_scatter(mlist, [ptr + csum - 1], packed, mask=m)
        return ptr + plsc.all_reduce_population_count(m)[0]

    n_match = lax.fori_loop(0, NCHUNKS_IDX, scan_step, jnp.int32(0))

    oflow = idx_v

    def zero_step(k, _):
        counts[pl.ds(pl.multiple_of(k * 16, 16), 16)] = zeros16
        return 0

    lax.fori_loop(0, CNT_PAD // 16, zero_step, 0)

    dyn_write(mlist, n_match, jnp.int32(127 << 22))

    def place_step(jj, n_over):
        j = 2 * jj
        v0 = dyn_read(mlist, j)
        v1 = dyn_read(mlist, j + 1)
        t0 = v0 >> 22
        t1 = v1 >> 22
        pos0 = dyn_read(counts, t0)
        pos1 = dyn_read(counts, t1) + (t0 == t1).astype(jnp.int32)
        dyn_write(counts, t0, pos0 + 1)
        dyn_write(counts, t1, pos1 + 1)
        ov0 = pos0 >= CAP
        ov1 = pos1 >= CAP

        @pl.when(jnp.logical_not(ov0))
        def _():
            dyn_write(glist2, t0 * CAP + pos0, v0)

        @pl.when(jnp.logical_not(ov1))
        def _():
            dyn_write(glist2, t1 * CAP + pos1, v1)

        @pl.when(ov0)
        def _():
            dyn_write(oflow, n_over, v0)

        @pl.when(ov1)
        def _():
            dyn_write(oflow, n_over + ov0.astype(jnp.int32), v1)

        return n_over + ov0.astype(jnp.int32) + ov1.astype(jnp.int32)

    n_over = lax.fori_loop(0, (n_match + 1) >> 1, place_step, jnp.int32(0))

    def flush(pad_from):
        dummy_base = B + wid * ROWCHUNK
        for kk in range(ROWCHUNK // 16):
            pos = kk * 16 + lanes
            bvals = blist[pl.ds(kk * 16, 16)]
            bvals = jnp.where(pos >= pad_from, dummy_base + pos, bvals)
            plsc.store_scatter(b2d, [zeros16, pos], bvals)
        pltpu.async_copy(rowbuf, out_hbm.at[b2d.at[0]], sem_s).wait()

    def emit_row(v, slot, gate, buf):
        col = (v >> 14) & (SBW - 1)
        bval = v & 16383
        gmask = jnp.broadcast_to(gate, (16,))
        csplat = jnp.broadcast_to(col, (16,))
        slotv = jnp.broadcast_to(slot, (16,))
        for c0 in range(0, D, 16):
            vals = plsc.load_gather(buf, [c0 + lanes, csplat], mask=gmask)
            plsc.store_scatter(rowbuf, [slotv, c0 + lanes], vals,
                               mask=gmask)
        plsc.store_scatter(blist, [slotv],
                           jnp.broadcast_to(bval, (16,)),
                           mask=gmask & (lanes == 0))

        @pl.when(gate & (slot == ROWCHUNK - 1))
        def _():
            flush(ROWCHUNK)

    def proc(tc_rel, buf, out_cnt):
        base = tc_rel * CAP
        n_bucket = jnp.minimum(dyn_read(counts, tc_rel), CAP)

        def match_step(j, cnt):
            v = dyn_read(glist2, base + j)
            emit_row(v, cnt % ROWCHUNK, jnp.bool_(True), buf)
            return cnt + 1

        out_cnt = lax.fori_loop(0, n_bucket, match_step, out_cnt)

        def over_step(j, cnt):
            v = dyn_read(oflow, j)
            mine = (v >> 22) == tc_rel
            emit_row(v, cnt % ROWCHUNK, mine, buf)
            return cnt + mine.astype(jnp.int32)

        return lax.fori_loop(0, n_over, over_step, out_cnt)

    def sweep_step(t, out_cnt):
        def body(cur, cur_sem, cnt):
            pltpu.make_async_copy(
                tt_hbm.at[:, pl.ds(0, SBW)], cur, cur_sem).wait()
            cnt = proc(t, cur, cnt)

            @pl.when(t + NBUF < n_sweep)
            def _():
                fire(t + NBUF, cur, cur_sem)

            return cnt

        return lax.switch(
            t % NBUF,
            [
                lambda cnt: body(buf0, sem0, cnt),
                lambda cnt: body(buf1, sem1, cnt),
                lambda cnt: body(buf2, sem2, cnt),
                lambda cnt: body(buf3, sem3, cnt),
            ],
            out_cnt,
        )

    out_cnt = lax.fori_loop(0, n_sweep, sweep_step, jnp.int32(0))

    @pl.when(has_tail)
    def _():
        pltpu.make_async_copy(
            tt_hbm.at[:, pl.ds(0, 128)], tailbuf, semt).wait()

    out_cnt = lax.cond(has_tail,
                       lambda: proc(n_sweep, tailbuf, out_cnt),
                       lambda: out_cnt)

    @pl.when(out_cnt % ROWCHUNK != 0)
    def _():
        flush(out_cnt % ROWCHUNK)


@jax.jit
def kernel(indices, table):
    idx = indices.astype(jnp.int32)
    tail = jnp.pad(table[TAIL_START:, :].T, ((0, 0), (0, 128 - TAIL_W)))
    mesh = plsc.VectorSubcoreMesh(
        core_axis_name="c", subcore_axis_name="s",
        num_cores=NUM_CORES, num_subcores=NUM_SUBCORES,
    )
    run = pl.kernel(
        _sweep_body,
        out_type=jax.ShapeDtypeStruct((PAD_ROWS, 128), jnp.float32),
        mesh=mesh,
        scratch_types=[
            pltpu.VMEM((B + 16,), jnp.int32),
            pltpu.VMEM((B + 16,), jnp.int32),
            pltpu.VMEM((CNT_PAD,), jnp.int32),
            pltpu.VMEM((CNT_PAD * CAP,), jnp.int32),
            pltpu.VMEM((D, SBW), jnp.float32),
            pltpu.VMEM((D, SBW), jnp.float32),
            pltpu.VMEM((D, SBW), jnp.float32),
            pltpu.VMEM((D, SBW), jnp.float32),
            pltpu.VMEM((D, 128), jnp.float32),
            pltpu.VMEM((ROWCHUNK, 128), jnp.float32),
            pltpu.VMEM((ROWCHUNK,), jnp.int32),
            pltpu.VMEM((1, ROWCHUNK), jnp.int32),
            pltpu.SemaphoreType.DMA,
            pltpu.SemaphoreType.DMA,
            pltpu.SemaphoreType.DMA,
            pltpu.SemaphoreType.DMA,
            pltpu.SemaphoreType.DMA,
            pltpu.SemaphoreType.DMA,
        ],
        compiler_params=pltpu.CompilerParams(
            use_tc_tiling_on_sc=True, needs_layout_passes=False),
    )
    out_pad = run(table.T, idx, tail)
    return out_pad[:B, :D]
